# baseline (speedup 1.0000x reference)
import functools

import jax
import jax.numpy as jnp
from jax import lax
from jax.experimental import pallas as pl
from jax.experimental.pallas import tpu as pltpu
from jax.experimental.pallas import tpu_sc as plsc

_NUM_CLASSES = 1000000
_HIDDEN = 64
_BATCH = 16384
_DEFAULT = -1

_NC, _NS, _L = 2, 16, 16
_NW = _NC * _NS
_BPW = _BATCH // _NW
_CHUNK = 128
_NCHUNK = _BPW // _CHUNK


def _make_kernel():
    mesh = plsc.VectorSubcoreMesh(core_axis_name="c", subcore_axis_name="s")

    @functools.partial(
        pl.kernel,
        mesh=mesh,
        compiler_params=pltpu.CompilerParams(use_tc_tiling_on_sc=False),
        out_type=jax.ShapeDtypeStruct((_BATCH, _HIDDEN), jnp.float32),
        scratch_types=[
            pltpu.VMEM((_BPW,), jnp.int32),
            pltpu.VMEM((_BPW, _HIDDEN), jnp.float32),
            pltpu.SemaphoreType.DMA,
            pltpu.SemaphoreType.DMA,
        ],
    )
    def k(labels_hbm, table_hbm, out_hbm, idx_v, rows_v, gsem, wsem):
        wid = lax.axis_index("s") * _NC + lax.axis_index("c")
        base = wid * _BPW
        pltpu.sync_copy(labels_hbm.at[pl.ds(base, _BPW)], idx_v)
        for i in range(_BPW // _L):
            v = idx_v[pl.ds(i * _L, _L)]
            idx_v[pl.ds(i * _L, _L)] = jnp.where(v == _DEFAULT, _NUM_CLASSES, v)
        gathers = []
        for c in range(_NCHUNK):
            gathers.append(
                pltpu.async_copy(
                    table_hbm.at[idx_v.at[pl.ds(c * _CHUNK, _CHUNK)]],
                    rows_v.at[pl.ds(c * _CHUNK, _CHUNK)],
                    gsem,
                )
            )
        writes = []
        for c, g in enumerate(gathers):
            g.wait()
            writes.append(
                pltpu.async_copy(
                    rows_v.at[pl.ds(c * _CHUNK, _CHUNK)],
                    out_hbm.at[pl.ds(base + c * _CHUNK, _CHUNK)],
                    wsem,
                )
            )
        for w in writes:
            w.wait()

    return k


_gather = _make_kernel()


def kernel(labels, embedding_table):
    return _gather(labels.astype(jnp.int32), embedding_table)
